# baseline (speedup 1.0000x reference)
import functools

import jax
import jax.numpy as jnp
from jax import lax
from jax.experimental import pallas as pl
from jax.experimental.pallas import tpu as pltpu
from jax.experimental.pallas import tpu_sc as plsc

_NUM_CORES = 2
_NUM_SUBCORES = 16
_NW = _NUM_CORES * _NUM_SUBCORES


def _sc_bcast_body(chunk, offset, rows, table_hbm, out_hbm, buf):
    batch = out_hbm.shape[0]
    rows_per_w = rows // _NW
    wid = lax.axis_index("s") * _NUM_CORES + lax.axis_index("c")
    base = wid * rows_per_w
    for c in range(rows_per_w // chunk):
        r0 = base + c * chunk
        pltpu.sync_copy(table_hbm.at[pl.ds(offset + r0, chunk)], buf)
        for b in range(batch):
            pltpu.sync_copy(buf, out_hbm.at[b, pl.ds(r0, chunk)])


def _sc_bcast(table, batch, offset, rows):
    dim = table.shape[1]
    rows_per_w = rows // _NW
    chunk = rows_per_w
    while chunk * dim * 4 > 480 * 1024 or chunk % 8:
        chunk //= 2
    assert chunk % 8 == 0 and rows_per_w % chunk == 0
    mesh = plsc.VectorSubcoreMesh(
        core_axis_name="c", subcore_axis_name="s",
        num_cores=_NUM_CORES, num_subcores=_NUM_SUBCORES)
    return pl.kernel(
        functools.partial(_sc_bcast_body, chunk, offset, rows), mesh=mesh,
        out_type=jax.ShapeDtypeStruct((batch, rows, dim), table.dtype),
        scratch_types=[pltpu.VMEM((chunk, dim), table.dtype)],
    )(table)


def _tc_bcast_body(t_ref, o_ref):
    o_ref[...] = jnp.broadcast_to(t_ref[...][None], o_ref.shape)


def _tc_bcast(table, batch, bs, nsteps):
    dim = table.shape[1]
    return pl.pallas_call(
        _tc_bcast_body,
        grid=(nsteps,),
        in_specs=[pl.BlockSpec((bs, dim), lambda s: (s, 0))],
        out_specs=pl.BlockSpec((batch, bs, dim), lambda s: (0, s, 0)),
        out_shape=jax.ShapeDtypeStruct((batch, nsteps * bs, dim), table.dtype),
    )(table)


def kernel(x, table):
    batch = x.shape[0]
    seq, dim = table.shape
    k = 4864
    tc_part = _tc_bcast(table, batch, bs=608, nsteps=k // 608)
    sc_part = _sc_bcast(table, batch, offset=k, rows=seq - k)
    return jnp.concatenate([tc_part, sc_part], axis=1)


# speedup vs baseline: 2.1312x; 2.1312x over previous
import functools

import jax
import jax.numpy as jnp
from jax import lax
from jax.experimental import pallas as pl
from jax.experimental.pallas import tpu as pltpu
from jax.experimental.pallas import tpu_sc as plsc

_NUM_CORES = 2
_NUM_SUBCORES = 16
_NW = _NUM_CORES * _NUM_SUBCORES


def _sc_tail_body(chunk, offset, rows, table_hbm, out_hbm, buf):
    batch = out_hbm.shape[0]
    rows_per_w = rows // _NW
    wid = lax.axis_index("s") * _NUM_CORES + lax.axis_index("c")
    base = offset + wid * rows_per_w
    for c in range(rows_per_w // chunk):
        r0 = base + c * chunk
        pltpu.sync_copy(table_hbm.at[pl.ds(r0, chunk)], buf)
        for b in range(batch):
            pltpu.sync_copy(buf, out_hbm.at[b, pl.ds(r0, chunk)])


def _sc_tail_bcast(table, batch, offset):
    seq, dim = table.shape
    rows = seq - offset
    rows_per_w = rows // _NW
    chunk = rows_per_w
    while chunk * dim * 4 > 480 * 1024 or chunk % 8:
        chunk //= 2
    assert chunk % 8 == 0 and rows_per_w % chunk == 0
    mesh = plsc.VectorSubcoreMesh(
        core_axis_name="c", subcore_axis_name="s",
        num_cores=_NUM_CORES, num_subcores=_NUM_SUBCORES)
    return pl.kernel(
        functools.partial(_sc_tail_body, chunk, offset, rows), mesh=mesh,
        out_type=jax.ShapeDtypeStruct((batch, seq, dim), table.dtype),
        scratch_types=[pltpu.VMEM((chunk, dim), table.dtype)],
    )(table)


def _tc_head_body(t_ref, _, o_ref):
    o_ref[...] = jnp.broadcast_to(t_ref[...][None], o_ref.shape)


def kernel(x, table):
    batch = x.shape[0]
    seq, dim = table.shape
    k = 5120
    bs = 512
    sc_out = _sc_tail_bcast(table, batch, offset=k)
    out = pl.pallas_call(
        _tc_head_body,
        grid=(k // bs,),
        in_specs=[
            pl.BlockSpec((bs, dim), lambda s: (s, 0)),
            pl.BlockSpec(memory_space=pl.ANY),
        ],
        out_specs=pl.BlockSpec((batch, bs, dim), lambda s: (0, s, 0)),
        out_shape=jax.ShapeDtypeStruct((batch, seq, dim), table.dtype),
        input_output_aliases={1: 0},
    )(table, sc_out)
    return out
